# baseline (speedup 1.0000x reference)
import functools
import math

import jax
import jax.numpy as jnp
from jax import lax
from jax.experimental import pallas as pl
from jax.experimental.pallas import tpu as pltpu
from jax.experimental.pallas import tpu_sc as plsc

_DIM = 128
_SCALE = math.sqrt(128.0)

_NC = 2
_NS = 16
_NW = _NC * _NS

_CHUNK = 128
_NBUF = 2


def _make_lookup(n_rows: int):
    assert n_rows % (_NW * _CHUNK) == 0
    per_w = n_rows // _NW
    n_chunks = per_w // _CHUNK
    assert n_chunks % _NBUF == 0
    n_outer = n_chunks // _NBUF
    mesh = plsc.VectorSubcoreMesh(
        core_axis_name="c", subcore_axis_name="s",
        num_cores=_NC, num_subcores=_NS,
    )

    @functools.partial(
        pl.kernel,
        out_type=jax.ShapeDtypeStruct((n_rows, _DIM), jnp.float32),
        mesh=mesh,
        scratch_types=[
            pltpu.VMEM((n_chunks, _CHUNK), jnp.int32),
            pltpu.VMEM((_NBUF, _CHUNK, _DIM), jnp.float32),
            pltpu.VMEM((_NBUF, _CHUNK, _DIM), jnp.float32),
            [pltpu.SemaphoreType.DMA] * _NBUF,
            [pltpu.SemaphoreType.DMA] * _NBUF,
        ],
    )
    def lookup(x_hbm, table_hbm, out_hbm, idx_v, gbuf, sbuf, gsems, ssems):
        wid = lax.axis_index("s") * _NC + lax.axis_index("c")
        pltpu.sync_copy(x_hbm.at[pl.ds(wid * n_chunks, n_chunks)], idx_v)
        row0 = wid * per_w

        def gather(j, b):
            return pltpu.make_async_copy(
                table_hbm.at[idx_v.at[j]], gbuf.at[b], gsems[b])

        def store(j, b):
            return pltpu.make_async_copy(
                sbuf.at[b], out_hbm.at[pl.ds(row0 + j * _CHUNK, _CHUNK)],
                ssems[b])

        def scale(b):
            @plsc.parallel_loop(0, _CHUNK, 1, unroll=4)
            def _(r):
                for k in range(_DIM // 16):
                    sl = pl.ds(k * 16, 16)
                    sbuf[b, r, sl] = gbuf[b, r, sl] * _SCALE

        for b in range(_NBUF):
            gather(b, b).start()

        def outer(g, carry):
            for b in range(_NBUF):
                i = g * _NBUF + b
                gather(i, b).wait()

                @pl.when(g > 0)
                def _():
                    store(i, b).wait()

                scale(b)
                store(i, b).start()

                @pl.when(g < n_outer - 1)
                def _():
                    gather(i + _NBUF, b).start()
            return carry

        lax.fori_loop(0, n_outer, outer, 0)
        for b in range(_NBUF):
            store(n_chunks - _NBUF + b, b).wait()

    return lookup


def kernel(x, table):
    orig_shape = x.shape
    n = 1
    for d in orig_shape:
        n *= d
    xf = x.reshape(_NW * (n // (_NW * _CHUNK)), _CHUNK).astype(jnp.int32)
    out = _make_lookup(n)(xf, table)
    return out.reshape(*orig_shape, _DIM)


# speedup vs baseline: 1.0011x; 1.0011x over previous
import functools
import math

import jax
import jax.numpy as jnp
from jax import lax
from jax.experimental import pallas as pl
from jax.experimental.pallas import tpu as pltpu
from jax.experimental.pallas import tpu_sc as plsc

_DIM = 128
_SCALE = math.sqrt(128.0)

_NC = 2
_NS = 16
_NW = _NC * _NS

_CHUNK = 128
_NBUF = 2


def _make_lookup(n_rows: int):
    assert n_rows % (_NW * _CHUNK) == 0
    per_w = n_rows // _NW
    n_chunks = per_w // _CHUNK
    assert n_chunks % _NBUF == 0
    n_outer = n_chunks // _NBUF
    mesh = plsc.VectorSubcoreMesh(
        core_axis_name="c", subcore_axis_name="s",
        num_cores=_NC, num_subcores=_NS,
    )

    @functools.partial(
        pl.kernel,
        out_type=jax.ShapeDtypeStruct((n_rows, _DIM), jnp.float32),
        mesh=mesh,
        scratch_types=[
            pltpu.VMEM((n_chunks, _CHUNK), jnp.int32),
            pltpu.VMEM((_NBUF, _CHUNK, _DIM), jnp.float32),
            pltpu.VMEM((_NBUF, _CHUNK, _DIM), jnp.float32),
            [pltpu.SemaphoreType.DMA] * _NBUF,
            [pltpu.SemaphoreType.DMA] * _NBUF,
        ],
    )
    def lookup(x_hbm, table_hbm, out_hbm, idx_v, gbuf, sbuf, gsems, ssems):
        wid = lax.axis_index("s") * _NC + lax.axis_index("c")
        pltpu.sync_copy(x_hbm.at[pl.ds(wid * n_chunks, n_chunks)], idx_v)
        row0 = wid * per_w

        def gather(j, b):
            return pltpu.make_async_copy(
                table_hbm.at[idx_v.at[j]], gbuf.at[b], gsems[b])

        def store(j, b):
            return pltpu.make_async_copy(
                sbuf.at[b], out_hbm.at[pl.ds(row0 + j * _CHUNK, _CHUNK)],
                ssems[b])

        def scale(b):
            @plsc.parallel_loop(0, _CHUNK, 1, unroll=8)
            def _(r):
                for k in range(_DIM // 16):
                    sl = pl.ds(k * 16, 16)
                    sbuf[b, r, sl] = gbuf[b, r, sl] * _SCALE

        for b in range(_NBUF):
            gather(b, b).start()

        def outer(g, carry):
            for b in range(_NBUF):
                i = g * _NBUF + b
                gather(i, b).wait()

                @pl.when(g > 0)
                def _():
                    store(i, b).wait()

                scale(b)
                store(i, b).start()

                @pl.when(g < n_outer - 1)
                def _():
                    gather(i + _NBUF, b).start()
            return carry

        lax.fori_loop(0, n_outer, outer, 0)
        for b in range(_NBUF):
            store(n_chunks - _NBUF + b, b).wait()

    return lookup


def kernel(x, table):
    orig_shape = x.shape
    n = 1
    for d in orig_shape:
        n *= d
    xf = x.reshape(_NW * (n // (_NW * _CHUNK)), _CHUNK).astype(jnp.int32)
    out = _make_lookup(n)(xf, table)
    return out.reshape(*orig_shape, _DIM)
